# baseline (speedup 1.0000x reference)
import functools

import numpy as np

import jax
import jax.numpy as jnp
from jax import lax
from jax.experimental import pallas as pl
from jax.experimental.pallas import tpu as pltpu


_LOG2E = 1.4426950408889634
_LN2 = 0.6931471805599453


def _tri_kernel(ti_ref, tj_ref, xa_ref, xb_ref, rs_ref, cs_ref, ps_ref, *,
                t, c0, half_tiles):
    c = pl.program_id(0)
    l = pl.program_id(1)
    ti = ti_ref[c, l]
    tj = tj_ref[c, l]

    xa = xa_ref[...]
    xb = xb_ref[...]
    dn = (((1,), (1,)), ((), ()))
    c0f = jnp.float32(c0)

    part = None
    cw = 256 if t % 256 == 0 else t
    for k8 in range(max(1, t // cw)):
        sk = lax.dot_general(xa, xb[k8 * cw:(k8 + 1) * cw, :], dn,
                             preferred_element_type=jnp.float32)
        for k2 in range(max(1, cw // 128)):
            col = k8 * cw + k2 * 128
            ek = jnp.exp2(sk[:, k2 * 128:(k2 + 1) * 128] - c0f)
            part = ek if part is None else part + ek
            cs_ref[0, :, col:col + 128] = ek.sum(axis=0, keepdims=True)

    rs_ref[0, :, :] = part.T.sum(axis=0, keepdims=True)
    ps_ref[0, :, :] = jnp.zeros((1, t), jnp.float32)

    def _diag_vec():
        prod = xa.astype(jnp.float32) * xb.astype(jnp.float32)
        dp = None
        for k in range(max(1, xa.shape[1] // 128)):
            blk = prod[:, k * 128:(k + 1) * 128]
            dp = blk if dp is None else dp + blk
        return dp.T.sum(axis=0, keepdims=True)

    @pl.when(ti == tj)
    def _diag():
        rs_ref[0, :, :] -= jnp.exp2(_diag_vec() - c0f)
        cs_ref[0, :, :] = jnp.zeros((1, t), jnp.float32)

    @pl.when(tj == ti + half_tiles)
    def _pos():
        ps_ref[0, :, :] = _diag_vec()


def _combine_kernel(rs_ref, cs_ref, ps_ref, arow_ref, acol_ref, out_ref, *,
                    n_steps, t, inv_temp):
    rs = rs_ref[...].reshape(n_steps, t)
    cs = cs_ref[...].reshape(n_steps, t)
    ps = ps_ref[...].reshape(n_steps, t)
    arow = arow_ref[...]
    acol = acol_ref[...]
    hi = lax.Precision.HIGHEST
    den = (lax.dot_general(arow, rs, (((1,), (0,)), ((), ())), precision=hi) +
           lax.dot_general(acol, cs, (((1,), (0,)), ((), ())), precision=hi))
    pos = (lax.dot_general(arow, ps, (((1,), (0,)), ((), ())), precision=hi) +
           lax.dot_general(acol, ps, (((1,), (0,)), ((), ())), precision=hi))
    out_ref[...] = inv_temp + jnp.log(den) - pos * jnp.float32(_LN2)


def _ntxent_sym(anchor, pos, temperature=0.1, t=2048):
    b, d = anchor.shape
    two_b = 2 * b
    nb = two_b // t
    assert two_b % t == 0 and b % t == 0 and nb % 2 == 0
    half_tiles = b // t
    inv_temp = float(1.0 / temperature)

    pairs = [(i, j) for i in range(nb) for j in range(i, nb)]
    n_pairs = len(pairs)
    n_cores = 2
    assert n_pairs % n_cores == 0
    n_per_core = n_pairs // n_cores
    ti_arr = np.zeros((n_cores, n_per_core), np.int32)
    tj_arr = np.zeros((n_cores, n_per_core), np.int32)
    for g, (i, j) in enumerate(pairs):
        ti_arr[g % n_cores, g // n_cores] = i
        tj_arr[g % n_cores, g // n_cores] = j
    arow = np.zeros((nb, n_pairs), np.float32)
    acol = np.zeros((nb, n_pairs), np.float32)
    for g, (i, j) in enumerate(pairs):
        arow[i, g] = 1.0
        if i != j:
            acol[j, g] = 1.0

    gamma = float(inv_temp * _LOG2E) ** 0.5
    x = (jnp.concatenate([anchor, pos], axis=0) * gamma).astype(
        jnp.float8_e4m3fn)

    tri = functools.partial(_tri_kernel, t=t, c0=inv_temp * _LOG2E,
                            half_tiles=half_tiles)
    grid = (n_cores, n_per_core)

    def _out_map(c, l, ti_m, tj_m):
        return (l * n_cores + c, 0, 0)

    rs, cs, ps = pl.pallas_call(
        tri,
        grid_spec=pltpu.PrefetchScalarGridSpec(
            num_scalar_prefetch=2,
            grid=grid,
            in_specs=[
                pl.BlockSpec((t, d), lambda c, l, ti_m, tj_m: (ti_m[c, l], 0)),
                pl.BlockSpec((t, d), lambda c, l, ti_m, tj_m: (tj_m[c, l], 0)),
            ],
            out_specs=[
                pl.BlockSpec((1, 1, t), _out_map),
                pl.BlockSpec((1, 1, t), _out_map),
                pl.BlockSpec((1, 1, t), _out_map),
            ],
        ),
        out_shape=[
            jax.ShapeDtypeStruct((n_pairs, 1, t), jnp.float32),
            jax.ShapeDtypeStruct((n_pairs, 1, t), jnp.float32),
            jax.ShapeDtypeStruct((n_pairs, 1, t), jnp.float32),
        ],
        compiler_params=pltpu.CompilerParams(
            dimension_semantics=("parallel", "arbitrary")),
    )(jnp.asarray(ti_arr), jnp.asarray(tj_arr), x, x)

    comb = functools.partial(_combine_kernel, n_steps=n_pairs, t=t,
                             inv_temp=inv_temp)
    row_losses = pl.pallas_call(
        comb,
        out_shape=jax.ShapeDtypeStruct((nb, t), jnp.float32),
    )(rs, cs, ps, jnp.asarray(arow), jnp.asarray(acol))
    return jnp.mean(row_losses)


def kernel(anchor, pos):
    return _ntxent_sym(anchor, pos, temperature=0.1, t=2048)


# speedup vs baseline: 1.0533x; 1.0533x over previous
import functools

import numpy as np

import jax
import jax.numpy as jnp
from jax import lax
from jax.experimental import pallas as pl
from jax.experimental.pallas import tpu as pltpu


_LOG2E = 1.4426950408889634
_LN2 = 0.6931471805599453


def _tri_kernel(ti_ref, tj_ref, xa_ref, xb_ref, rs_ref, cs_ref, ps_ref, *,
                t, c0, half_tiles):
    c = pl.program_id(0)
    l = pl.program_id(1)
    ti = ti_ref[c, l]
    tj = tj_ref[c, l]

    xa = xa_ref[...]
    xb = xb_ref[...]
    dn = (((1,), (1,)), ((), ()))
    s = lax.dot_general(xa, xb, dn,
                        preferred_element_type=jnp.float32)
    c0f = jnp.float32(c0)

    nk = max(1, t // 128)
    part = None
    for k in range(nk):
        ek = jnp.exp2(s[:, k * 128:(k + 1) * 128] - c0f)
        part = ek if k == 0 else part + ek
        cs_ref[0, :, k * 128:(k + 1) * 128] = ek.sum(axis=0, keepdims=True)

    rs_ref[0, :, :] = part.T.sum(axis=0, keepdims=True)
    ps_ref[0, :, :] = jnp.zeros((1, t), jnp.float32)

    @pl.when(ti == tj)
    def _diag():
        dmask = (lax.broadcasted_iota(jnp.int32, (t, t), 0) ==
                 lax.broadcasted_iota(jnp.int32, (t, t), 1))
        sd = jnp.where(dmask, s, 0.0).sum(axis=0, keepdims=True)
        rs_ref[0, :, :] -= jnp.exp2(sd - c0f)
        cs_ref[0, :, :] = jnp.zeros((1, t), jnp.float32)

    @pl.when(tj == ti + half_tiles)
    def _pos():
        dmask = (lax.broadcasted_iota(jnp.int32, (t, t), 0) ==
                 lax.broadcasted_iota(jnp.int32, (t, t), 1))
        ps_ref[0, :, :] = jnp.where(dmask, s, 0.0).sum(axis=0, keepdims=True)


def _combine_kernel(rs_ref, cs_ref, ps_ref, arow_ref, acol_ref, out_ref, *,
                    n_steps, t, inv_temp):
    rs = rs_ref[...].reshape(n_steps, t)
    cs = cs_ref[...].reshape(n_steps, t)
    ps = ps_ref[...].reshape(n_steps, t)
    arow = arow_ref[...]
    acol = acol_ref[...]
    hi = lax.Precision.HIGHEST
    den = (lax.dot_general(arow, rs, (((1,), (0,)), ((), ())), precision=hi) +
           lax.dot_general(acol, cs, (((1,), (0,)), ((), ())), precision=hi))
    pos = (lax.dot_general(arow, ps, (((1,), (0,)), ((), ())), precision=hi) +
           lax.dot_general(acol, ps, (((1,), (0,)), ((), ())), precision=hi))
    losses = inv_temp + jnp.log(den) - pos * jnp.float32(_LN2)
    nb_, t_ = losses.shape
    out_ref[...] = jnp.sum(losses, axis=1, keepdims=True).sum(
        axis=0, keepdims=True) * jnp.float32(1.0 / (nb_ * t_))


def _ntxent_sym(anchor, pos, temperature=0.1, t=2048):
    b, d = anchor.shape
    two_b = 2 * b
    nb = two_b // t
    assert two_b % t == 0 and b % t == 0 and nb % 2 == 0
    half_tiles = b // t
    inv_temp = float(1.0 / temperature)

    pairs = [(i, j) for i in range(nb) for j in range(i, nb)]
    n_pairs = len(pairs)
    n_cores = 2
    assert n_pairs % n_cores == 0
    n_per_core = n_pairs // n_cores
    ti_arr = np.zeros((n_cores, n_per_core), np.int32)
    tj_arr = np.zeros((n_cores, n_per_core), np.int32)
    for g, (i, j) in enumerate(pairs):
        ti_arr[g % n_cores, g // n_cores] = i
        tj_arr[g % n_cores, g // n_cores] = j
    arow = np.zeros((nb, n_pairs), np.float32)
    acol = np.zeros((nb, n_pairs), np.float32)
    for g, (i, j) in enumerate(pairs):
        arow[i, g] = 1.0
        if i != j:
            acol[j, g] = 1.0

    gamma = float(inv_temp * _LOG2E) ** 0.5
    x = (jnp.concatenate([anchor, pos], axis=0) * gamma).astype(
        jnp.float8_e4m3fn)

    tri = functools.partial(_tri_kernel, t=t, c0=inv_temp * _LOG2E,
                            half_tiles=half_tiles)
    grid = (n_cores, n_per_core)

    def _out_map(c, l, ti_m, tj_m):
        return (l * n_cores + c, 0, 0)

    rs, cs, ps = pl.pallas_call(
        tri,
        grid_spec=pltpu.PrefetchScalarGridSpec(
            num_scalar_prefetch=2,
            grid=grid,
            in_specs=[
                pl.BlockSpec((t, d), lambda c, l, ti_m, tj_m: (ti_m[c, l], 0)),
                pl.BlockSpec((t, d), lambda c, l, ti_m, tj_m: (tj_m[c, l], 0)),
            ],
            out_specs=[
                pl.BlockSpec((1, 1, t), _out_map),
                pl.BlockSpec((1, 1, t), _out_map),
                pl.BlockSpec((1, 1, t), _out_map),
            ],
        ),
        out_shape=[
            jax.ShapeDtypeStruct((n_pairs, 1, t), jnp.float32),
            jax.ShapeDtypeStruct((n_pairs, 1, t), jnp.float32),
            jax.ShapeDtypeStruct((n_pairs, 1, t), jnp.float32),
        ],
        compiler_params=pltpu.CompilerParams(
            dimension_semantics=("parallel", "arbitrary")),
    )(jnp.asarray(ti_arr), jnp.asarray(tj_arr), x, x)

    comb = functools.partial(_combine_kernel, n_steps=n_pairs, t=t,
                             inv_temp=inv_temp)
    loss = pl.pallas_call(
        comb,
        out_shape=jax.ShapeDtypeStruct((1, 1), jnp.float32),
    )(rs, cs, ps, jnp.asarray(arow), jnp.asarray(acol))
    return loss[0, 0]


def kernel(anchor, pos):
    return _ntxent_sym(anchor, pos, temperature=0.1, t=2048)
